# Initial kernel scaffold; baseline (speedup 1.0000x reference)
import jax
import jax.numpy as jnp
from jax.experimental import pallas as pl


def kernel(x_protein, x_class, ei_pos, ei_neg, ei_link, ei_ppi, mask, W_a_pos_rel, b_a_pos, W_a_pos_root, W_a_neg_rel, b_a_neg, W_a_neg_root, W_a_link_rel, b_a_link, W_a_link_root, W_a_ppi_rel, b_a_ppi, W_a_ppi_root, W_b_pos_rel, b_b_pos, W_b_pos_root, W_b_neg_rel, b_b_neg, W_b_neg_root, W_b_link_rel, b_b_link, W_b_link_root, W_b_ppi_rel, b_b_ppi, W_b_ppi_root, W_lin, b_lin):
    raise NotImplementedError("write your pallas kernel here")


# speedup vs baseline: 7.1572x; 7.1572x over previous
import functools

import jax
import jax.numpy as jnp
from jax import lax
from jax.experimental import pallas as pl
from jax.experimental.pallas import tpu as pltpu
from jax.experimental.pallas import tpu_sc as plsc

N = 10000
D = 128
H = 256
E = 160000
M = 16384
NCORES = 2
NSUB = 16
NW = NCORES * NSUB
EPW = E // NW
CH = 128
FULL = EPW // CH
TAIL = EPW - FULL * CH
NPAD = 10240
RPT = NPAD // NSUB
ZR = 40
OPW = M // NW

_mesh = plsc.VectorSubcoreMesh(core_axis_name="c", subcore_axis_name="s")


def _scatter_body(x_hbm, src_hbm, dst_hbm, out_hbm,
                  sidx, didx, rows, sidx_t, didx_t, rows_t, zbuf, acc, sem):
    c = lax.axis_index("c")
    s = lax.axis_index("s")
    wid = s * NCORES + c
    z16 = jnp.zeros((16,), jnp.float32)

    def _zrow(r, _):
        def _zcol(cc, _):
            zbuf[r, pl.ds(cc * 16, 16)] = z16
            return 0
        return lax.fori_loop(0, D // 16, _zcol, 0)

    lax.fori_loop(0, ZR, _zrow, 0)

    def _zacc(j, _):
        pltpu.sync_copy(zbuf, acc.at[pl.ds(s * RPT + j * ZR, ZR)])
        return 0

    lax.fori_loop(0, RPT // ZR, _zacc, 0)
    plsc.subcore_barrier()

    base = wid * EPW

    def _chunk(j, _):
        off = base + j * CH
        pltpu.sync_copy(src_hbm.at[pl.ds(off, CH)], sidx)
        pltpu.sync_copy(dst_hbm.at[pl.ds(off, CH)], didx)
        pltpu.async_copy(x_hbm.at[sidx], rows, sem).wait()
        pltpu.sync_copy(rows, acc.at[didx], add=True)
        return 0

    lax.fori_loop(0, FULL, _chunk, 0)

    offt = base + FULL * CH
    pltpu.sync_copy(src_hbm.at[pl.ds(offt, TAIL)], sidx_t)
    pltpu.sync_copy(dst_hbm.at[pl.ds(offt, TAIL)], didx_t)
    pltpu.async_copy(x_hbm.at[sidx_t], rows_t, sem).wait()
    pltpu.sync_copy(rows_t, acc.at[didx_t], add=True)

    plsc.subcore_barrier()
    pltpu.sync_copy(acc.at[pl.ds(s * RPT, RPT)], out_hbm.at[c, pl.ds(s * RPT, RPT)])


_scatter = pl.kernel(
    _scatter_body,
    out_type=jax.ShapeDtypeStruct((NCORES, NPAD, D), jnp.float32),
    mesh=_mesh,
    scratch_types=[
        pltpu.VMEM((CH,), jnp.int32),
        pltpu.VMEM((CH,), jnp.int32),
        pltpu.VMEM((CH, D), jnp.float32),
        pltpu.VMEM((TAIL,), jnp.int32),
        pltpu.VMEM((TAIL,), jnp.int32),
        pltpu.VMEM((TAIL, D), jnp.float32),
        pltpu.VMEM((ZR, D), jnp.float32),
        pltpu.VMEM_SHARED((NPAD, D), jnp.float32),
        pltpu.SemaphoreType.DMA,
    ],
)


def _head_body(g_hbm, m0_hbm, m1_hbm, out_hbm, gv, m0v, m1v, ov):
    c = lax.axis_index("c")
    s = lax.axis_index("s")
    wid = s * NCORES + c
    base = wid * OPW
    pltpu.sync_copy(g_hbm, gv)
    pltpu.sync_copy(m0_hbm.at[pl.ds(base, OPW)], m0v)
    pltpu.sync_copy(m1_hbm.at[pl.ds(base, OPW)], m1v)

    def _grp(k, _):
        i0 = m0v[pl.ds(k * 16, 16)]
        i1 = m1v[pl.ds(k * 16, 16)]
        v0 = plsc.load_gather(gv, [i0 * 2])
        v1 = plsc.load_gather(gv, [i1 * 2 + 1])
        x = v0 + v1
        ov[pl.ds(k * 16, 16)] = 1.0 / (1.0 + jnp.exp(-x))
        return 0

    lax.fori_loop(0, OPW // 16, _grp, 0)
    pltpu.sync_copy(ov, out_hbm.at[pl.ds(base, OPW)])


_head = pl.kernel(
    _head_body,
    out_type=jax.ShapeDtypeStruct((M,), jnp.float32),
    mesh=_mesh,
    compiler_params=pltpu.CompilerParams(needs_layout_passes=False),
    scratch_types=[
        pltpu.VMEM((2 * N,), jnp.float32),
        pltpu.VMEM((OPW,), jnp.int32),
        pltpu.VMEM((OPW,), jnp.int32),
        pltpu.VMEM((OPW,), jnp.float32),
    ],
)

BLK = 1000


def _mm1_body(p_ref, xp_ref, wrel_ref, wroot_ref, ba_ref, wbrel_ref,
              wbroot_ref, u_ref, r_ref):
    agg = p_ref[0] + p_ref[1]
    hp = jnp.dot(agg, wrel_ref[...], preferred_element_type=jnp.float32)
    hp = hp + jnp.dot(xp_ref[...], wroot_ref[...],
                      preferred_element_type=jnp.float32)
    hp = jnp.maximum(hp + ba_ref[...], 0.0)
    u_ref[...] = jnp.dot(hp, wbrel_ref[...], preferred_element_type=jnp.float32)
    r_ref[...] = jnp.dot(hp, wbroot_ref[...], preferred_element_type=jnp.float32)


def _mm1(P, xp, wrel, wroot, ba, wbrel, wbroot):
    grid = (N // BLK,)
    full = lambda i: (0, 0)
    return pl.pallas_call(
        _mm1_body,
        grid=grid,
        in_specs=[
            pl.BlockSpec((NCORES, BLK, D), lambda i: (0, i, 0)),
            pl.BlockSpec((BLK, D), lambda i: (i, 0)),
            pl.BlockSpec((D, H), full),
            pl.BlockSpec((D, H), full),
            pl.BlockSpec((1, H), full),
            pl.BlockSpec((H, D), full),
            pl.BlockSpec((H, D), full),
        ],
        out_specs=[
            pl.BlockSpec((BLK, D), lambda i: (i, 0)),
            pl.BlockSpec((BLK, D), lambda i: (i, 0)),
        ],
        out_shape=[
            jax.ShapeDtypeStruct((N, D), jnp.float32),
            jax.ShapeDtypeStruct((N, D), jnp.float32),
        ],
    )(P, xp, wrel, wroot, ba, wbrel, wbroot)


def _mm2_body(q_ref, r_ref, bb_ref, wl_ref, gb_ref, g_ref):
    h2 = q_ref[0] + q_ref[1] + r_ref[...] + bb_ref[...]
    g_ref[...] = jnp.dot(h2, wl_ref[...],
                         preferred_element_type=jnp.float32) + gb_ref[...]


def _mm2(Q, R, bb, wl, gb):
    grid = (N // BLK,)
    full = lambda i: (0, 0)
    return pl.pallas_call(
        _mm2_body,
        grid=grid,
        in_specs=[
            pl.BlockSpec((NCORES, BLK, D), lambda i: (0, i, 0)),
            pl.BlockSpec((BLK, D), lambda i: (i, 0)),
            pl.BlockSpec((1, D), full),
            pl.BlockSpec((D, 2), full),
            pl.BlockSpec((1, 2), full),
        ],
        out_specs=pl.BlockSpec((BLK, 2), lambda i: (i, 0)),
        out_shape=jax.ShapeDtypeStruct((N, 2), jnp.float32),
    )(Q, R, bb, wl, gb)


def kernel(x_protein, x_class, ei_pos, ei_neg, ei_link, ei_ppi, mask,
           W_a_pos_rel, b_a_pos, W_a_pos_root,
           W_a_neg_rel, b_a_neg, W_a_neg_root,
           W_a_link_rel, b_a_link, W_a_link_root,
           W_a_ppi_rel, b_a_ppi, W_a_ppi_root,
           W_b_pos_rel, b_b_pos, W_b_pos_root,
           W_b_neg_rel, b_b_neg, W_b_neg_root,
           W_b_link_rel, b_b_link, W_b_link_root,
           W_b_ppi_rel, b_b_ppi, W_b_ppi_root,
           W_lin, b_lin):
    src = ei_ppi[0]
    dst = ei_ppi[1]
    P = _scatter(x_protein, src, dst)
    U, R = _mm1(P, x_protein, W_a_ppi_rel, W_a_ppi_root,
                b_a_ppi.reshape(1, H), W_b_ppi_rel, W_b_ppi_root)
    Q = _scatter(U, src, dst)
    wl = jnp.concatenate([W_lin[:D], W_lin[D:]], axis=1)
    gb = jnp.stack([b_lin[0], jnp.float32(0.0)]).reshape(1, 2)
    G = _mm2(Q, R, b_b_ppi.reshape(1, D), wl, gb)
    mt = mask.T
    out = _head(G.reshape(-1), mt[0], mt[1])
    return out.reshape(M, 1)
